# baseline (speedup 1.0000x reference)
import functools
import math

import jax
import jax.numpy as jnp
from jax import lax
from jax.experimental import pallas as pl
from jax.experimental.pallas import tpu as pltpu
from jax.experimental.pallas import tpu_sc as plsc

B, L, D = 1024, 200, 128
VE, VA, VV = 64, 32, 16
TB = 64
SPLIT = 1024

NDAYS = 3656
WALL_ROWS = 4096
T_BASE = 128
PE_BASE = 128 + NDAYS
NW = 32
TOK_SUB = 25


def _wall_body(wcat_ref, wall_ref):
    lane1 = jax.lax.broadcasted_iota(jnp.int32, (1, D), 1)
    j2 = ((lane1 // 2) * 2).astype(jnp.float32)
    div = jnp.exp(j2 * (-math.log(10000.0) / D))
    phase = (lane1 % 2).astype(jnp.float32) * (math.pi / 2.0)
    day = jax.lax.broadcasted_iota(jnp.int32, (NDAYS, D), 0).astype(jnp.float32)
    tday = jnp.sin(day * div + phase)
    pos = jax.lax.broadcasted_iota(jnp.int32, (L, D), 0).astype(jnp.float32)
    pe = jnp.sin(pos * div + phase)
    pad = jnp.zeros((WALL_ROWS - PE_BASE - L, D), jnp.float32)
    wall_ref[...] = jnp.concatenate([wcat_ref[...], tday, pe, pad], axis=0)


def _build_wall(wcat, interpret=False):
    return pl.pallas_call(
        _wall_body,
        in_specs=[pl.BlockSpec((D, D), lambda: (0, 0))],
        out_specs=pl.BlockSpec((WALL_ROWS, D), lambda: (0, 0)),
        out_shape=jax.ShapeDtypeStruct((WALL_ROWS, D), jnp.float32),
        interpret=interpret,
    )(wcat)


def _make_sc_kernel(S):
    NT = S * L
    BROWS_W = S // NW

    mesh = plsc.VectorSubcoreMesh(core_axis_name="c", subcore_axis_name="s")

    @functools.partial(
        pl.kernel, mesh=mesh,
        out_type=jax.ShapeDtypeStruct((NT, D), jnp.float32),
        scratch_types=[
            pltpu.VMEM((8, 128), jnp.int32),
            pltpu.VMEM((128, D), jnp.float32),
            pltpu.VMEM((128, D), jnp.float32),
            pltpu.VMEM((L, D), jnp.float32),
            pltpu.VMEM((L, D), jnp.float32),
            pltpu.SemaphoreType.DMA,
            pltpu.SemaphoreType.DMA,
        ],
    )
    def sc_k(wall_hbm, ids_hbm, out_hbm, idx_v, rows_v0, rows_v1, pe_v, ob_v,
             sem0, sem1):
        wid = lax.axis_index("s") * 2 + lax.axis_index("c")
        pltpu.sync_copy(wall_hbm.at[pl.ds(PE_BASE, L)], pe_v)
        bufs = (rows_v0, rows_v1)
        sems = (sem0, sem1)

        def brow_body(brow, _):
            row = wid * BROWS_W + brow
            pltpu.sync_copy(ids_hbm.at[pl.ds(row * 8, 8)], idx_v)
            handles = [None, None]
            handles[0] = pltpu.async_copy(wall_hbm.at[idx_v.at[0]], bufs[0],
                                          sems[0])
            for gg in range(8):
                cur = gg % 2
                if gg < 7:
                    handles[1 - cur] = pltpu.async_copy(
                        wall_hbm.at[idx_v.at[gg + 1]], bufs[1 - cur],
                        sems[1 - cur])
                handles[cur].wait()
                rows_v = bufs[cur]
                l0 = gg * TOK_SUB

                def tok_body(i, _):
                    for kk in range(8):
                        col = pl.ds(kk * 16, 16)
                        acc = (rows_v[4 * i, col] + rows_v[4 * i + 1, col]
                               + rows_v[4 * i + 2, col] + rows_v[4 * i + 3, col]
                               + pe_v[l0 + i, col])
                        ob_v[l0 + i, col] = acc
                    return 0

                lax.fori_loop(0, TOK_SUB, tok_body, 0, unroll=5)
            pltpu.sync_copy(ob_v, out_hbm.at[pl.ds(row * L, L)])
            return 0

        lax.fori_loop(0, BROWS_W, brow_body, 0)

    return sc_k


def _embed_body(e_ref, a_ref, v_ref, d_ref, wcat_ref, o_ref, tt_ref, pe_ref):
    lane1 = jax.lax.broadcasted_iota(jnp.int32, (1, D), 1)
    j2 = ((lane1 // 2) * 2).astype(jnp.float32)
    div = jnp.exp(j2 * (-math.log(10000.0) / D))
    phase = (lane1 % 2).astype(jnp.float32) * (math.pi / 2.0)

    @pl.when(pl.program_id(0) == 0)
    def _build_tables():
        h = jax.lax.broadcasted_iota(jnp.int32, (64, D), 0).astype(jnp.float32)
        arg_a = (h * 64.0) * div
        arg_b = h * div + phase
        top = jnp.concatenate(
            [jnp.sin(arg_a), jnp.cos(arg_a), jnp.zeros((64, 2 * D), jnp.float32)],
            axis=1)
        bot = jnp.concatenate(
            [jnp.zeros((64, 2 * D), jnp.float32), jnp.cos(arg_b), jnp.sin(arg_b)],
            axis=1)
        tt_ref[...] = jnp.concatenate([top, bot], axis=0).astype(jnp.bfloat16)
        pos = jax.lax.broadcasted_iota(jnp.int32, (L, D), 0).astype(jnp.float32)
        pe_ref[...] = jnp.sin(pos * div + phase)

    e = e_ref[...].astype(jnp.int16)[..., None]
    a = a_ref[...].astype(jnp.int16)[..., None]
    v = v_ref[...].astype(jnp.int16)[..., None]
    d = d_ref[...][..., None]
    lane16 = jax.lax.broadcasted_iota(jnp.int16, (TB, L, D), 2)
    m = ((lane16 == e) | (lane16 == a + VE) | (lane16 == v + (VE + VA))
         ).astype(jnp.bfloat16).reshape(TB * L, D)
    eav = jnp.dot(m, wcat_ref[...].astype(jnp.bfloat16),
                  preferred_element_type=jnp.float32)

    lane = jax.lax.broadcasted_iota(jnp.int32, (TB, L, D), 2)
    hi = d >> 6
    lo = d & 63
    ohl = ((lane == hi) | (lane == lo + 64)).astype(jnp.bfloat16).reshape(TB * L, D)
    r = jnp.dot(ohl, tt_ref[...], preferred_element_type=jnp.float32)
    time_emb = (r[:, 0:D] * r[:, 2 * D:3 * D]
                + r[:, D:2 * D] * r[:, 3 * D:4 * D])

    out = (eav + time_emb).reshape(TB, L, D) + pe_ref[...][None, :, :]
    o_ref[...] = out


def _run_tc(entity_id, attribute_id, value_id, days_since_tpx, wcat,
            nrows, interpret=False):
    grid = (nrows // TB,)
    idx_spec = pl.BlockSpec((TB, L), lambda i: (i, 0))
    return pl.pallas_call(
        _embed_body,
        grid=grid,
        in_specs=[idx_spec, idx_spec, idx_spec, idx_spec,
                  pl.BlockSpec((D, D), lambda i: (0, 0))],
        out_specs=pl.BlockSpec((TB, L, D), lambda i: (i, 0, 0)),
        out_shape=jax.ShapeDtypeStruct((nrows, L, D), jnp.float32),
        scratch_shapes=[pltpu.VMEM((D, 4 * D), jnp.bfloat16),
                        pltpu.VMEM((L, D), jnp.float32)],
        compiler_params=pltpu.CompilerParams(
            dimension_semantics=("arbitrary",),
        ),
        interpret=interpret,
    )(entity_id, attribute_id, value_id, days_since_tpx, wcat)


@functools.partial(jax.jit, static_argnames=("interpret",))
def kernel(entity_id, attribute_id, value_id, days_since_tpx,
           W_entity, W_attribute, W_value, interpret=False):
    wcat = jnp.zeros((D, D), jnp.float32)
    wcat = jax.lax.dynamic_update_slice(wcat, W_entity, (0, 0))
    wcat = jax.lax.dynamic_update_slice(wcat, W_attribute, (VE, 0))
    wcat = jax.lax.dynamic_update_slice(wcat, W_value, (VE + VA, 0))

    parts = []
    if SPLIT > 0:
        wall = _build_wall(wcat, interpret=interpret)
        ids = jnp.stack(
            [entity_id[:SPLIT].reshape(-1),
             attribute_id[:SPLIT].reshape(-1) + VE,
             value_id[:SPLIT].reshape(-1) + (VE + VA),
             days_since_tpx[:SPLIT].reshape(-1) + T_BASE],
            axis=-1).astype(jnp.int32).reshape(8 * SPLIT, 4 * TOK_SUB)
        ids = jnp.pad(ids, ((0, 0), (0, 128 - 4 * TOK_SUB)),
                      constant_values=WALL_ROWS - 1)
        sc_out = _make_sc_kernel(SPLIT)(wall, ids)
        parts.append(sc_out.reshape(SPLIT, L, D))
    if SPLIT < B:
        parts.append(_run_tc(
            entity_id[SPLIT:], attribute_id[SPLIT:], value_id[SPLIT:],
            days_since_tpx[SPLIT:], wcat, B - SPLIT, interpret=interpret))
    return parts[0] if len(parts) == 1 else jnp.concatenate(parts, axis=0)
